# Initial kernel scaffold; baseline (speedup 1.0000x reference)
import jax
import jax.numpy as jnp
from jax.experimental import pallas as pl


def kernel(input_embs, input_bt):
    raise NotImplementedError("write your pallas kernel here")


# speedup vs baseline: 2.7111x; 2.7111x over previous
import functools

import jax
import jax.numpy as jnp
from jax import lax
from jax.experimental import pallas as pl
from jax.experimental.pallas import tpu as pltpu
from jax.experimental.pallas import tpu_sc as plsc

BTYPES = 4
L = 200
H = 64
NLANE = 16
NSUB = 16
SLOTS = 256
NBUF = BTYPES * SLOTS
CHK = 128


def _mesh():
    return plsc.VectorSubcoreMesh(
        core_axis_name="c", subcore_axis_name="s",
        num_cores=2, num_subcores=NSUB)


def _sc_counts(bt_t):
    chunk = bt_t.shape[1]
    sub_len = chunk // NLANE

    @functools.partial(
        pl.kernel,
        out_type=jax.ShapeDtypeStruct((NSUB, BTYPES, NLANE), jnp.int32),
        mesh=_mesh(),
        scratch_types=[
            pltpu.VMEM((chunk,), jnp.int32),
            pltpu.VMEM((BTYPES, NLANE), jnp.int32),
        ],
    )
    def k(bt_hbm, cnt_hbm, bt_v, cnt4_v):
        cid = lax.axis_index("c")
        wid = lax.axis_index("s")

        @pl.when(cid == 0)
        def _():
            zero = jnp.zeros((NLANE,), jnp.int32)
            one = jnp.full((NLANE,), 1, jnp.int32)
            pltpu.sync_copy(bt_hbm.at[wid], bt_v)

            def p1(i, acc):
                v = bt_v[pl.ds(i * NLANE, NLANE)]
                return tuple(acc[b] + jnp.where(v == (b + 1), one, zero)
                             for b in range(BTYPES))
            acc = lax.fori_loop(0, sub_len, p1, (zero,) * BTYPES)
            for b in range(BTYPES):
                cnt4_v[b] = acc[b]
            pltpu.sync_copy(cnt4_v, cnt_hbm.at[wid])

    return k(bt_t)


def _tc_prefix(c256):
    nsc = NSUB * NLANE

    def body(c_ref, st_ref, aux_ref):
        cf = c_ref[...].astype(jnp.float32)
        ri = lax.broadcasted_iota(jnp.int32, (nsc, nsc), 0)
        ci = lax.broadcasted_iota(jnp.int32, (nsc, nsc), 1)
        tmat = jnp.where(ri < ci, 1.0, 0.0)
        ex = jnp.dot(cf, tmat, preferred_element_type=jnp.float32)
        st_ref[...] = ex.astype(jnp.int32)
        tot = jnp.sum(c_ref[...], axis=1, keepdims=True)
        thr = jnp.maximum(tot - L, 0)
        off = (L - 1) - tot
        aux_ref[...] = jnp.concatenate(
            [jnp.broadcast_to(thr, (BTYPES, NLANE)),
             jnp.broadcast_to(off, (BTYPES, NLANE)),
             jnp.broadcast_to(tot, (BTYPES, NLANE))], axis=0)

    return pl.pallas_call(
        body,
        out_shape=(
            jax.ShapeDtypeStruct((BTYPES, nsc), jnp.int32),
            jax.ShapeDtypeStruct((3 * BTYPES, NLANE), jnp.int32),
        ),
    )(c256)


def _sc_select(bt_t, embs_flat, starts_t, aux):
    chunk = bt_t.shape[1]
    sub_len = chunk // NLANE
    epw = NBUF // NSUB
    nchunks = chunk // CHK
    vper = CHK // NLANE

    @functools.partial(
        pl.kernel,
        out_type=jax.ShapeDtypeStruct((NBUF, H), jnp.float32),
        mesh=_mesh(),
        compiler_params=pltpu.CompilerParams(use_tc_tiling_on_sc=False),
        scratch_types=[
            pltpu.VMEM((chunk,), jnp.int32),
            pltpu.VMEM((BTYPES, NLANE), jnp.int32),
            pltpu.VMEM((2 * BTYPES, NLANE), jnp.int32),
            pltpu.VMEM((BTYPES, chunk), jnp.int32),
            pltpu.VMEM((BTYPES, chunk), jnp.int32),
            pltpu.VMEM((NBUF,), jnp.int32),
            pltpu.VMEM((epw,), jnp.int32),
            pltpu.VMEM((epw,), jnp.int32),
            pltpu.VMEM((epw, H), jnp.float32),
            pltpu.VMEM_SHARED((NBUF,), jnp.int32),
            pltpu.SemaphoreType.DMA,
        ],
    )
    def k(bt_hbm, embs_hbm, st_hbm, aux_hbm, seq_hbm,
          bt_v, st_v, to_v, slot_v, val_v, zb_v, mg_v, idx_v, rows_v,
          msh, sem):
        cid = lax.axis_index("c")
        wid = lax.axis_index("s")

        @pl.when(cid == 0)
        def _():
            iota = lax.iota(jnp.int32, NLANE)
            zero = jnp.zeros((NLANE,), jnp.int32)
            one = jnp.full((NLANE,), 1, jnp.int32)

            for i in range(NBUF // NLANE):
                zb_v[pl.ds(i * NLANE, NLANE)] = zero

            @pl.when(wid == 0)
            def _():
                pltpu.sync_copy(zb_v, msh)

            pltpu.sync_copy(bt_hbm.at[wid], bt_v)
            pltpu.sync_copy(st_hbm.at[wid], st_v)
            pltpu.sync_copy(aux_hbm.at[pl.ds(0, 2 * BTYPES)], to_v)

            starts = [st_v[b] for b in range(BTYPES)]
            thr = [to_v[b] for b in range(BTYPES)]
            off = [to_v[BTYPES + b] for b in range(BTYPES)]
            pos_base = (wid * NLANE + iota) * sub_len + 1

            def p2(i, carry):
                v = bt_v[pl.ds(i * NLANE, NLANE)]
                pos1 = pos_base + i
                new = []
                for b in range(BTYPES):
                    mi = jnp.where(v == (b + 1), one, zero)
                    r = carry[b] + mi
                    ki = jnp.where(mi * r - thr[b] > zero, one, zero)
                    slot = jnp.maximum(r + off[b], zero) + b * SLOTS
                    val_v[b, pl.ds(i * NLANE, NLANE)] = ki * pos1
                    slot_v[b, pl.ds(i * NLANE, NLANE)] = slot
                    new.append(r)
                return tuple(new)

            lax.fori_loop(0, sub_len, p2, tuple(starts))

            plsc.subcore_barrier()

            def merge(j, carry):
                for b in range(BTYPES):
                    pltpu.sync_copy(
                        val_v.at[b, pl.ds(j * CHK, CHK)],
                        msh.at[slot_v.at[b, pl.ds(j * CHK, CHK)]],
                        add=True)
                return carry

            lax.fori_loop(0, nchunks, merge, 0)
            plsc.subcore_barrier()

            pltpu.sync_copy(msh.at[pl.ds(wid * epw, epw)], mg_v)
            for j in range(epw // NLANE):
                idx_v[pl.ds(j * NLANE, NLANE)] = jnp.maximum(
                    mg_v[pl.ds(j * NLANE, NLANE)] - one, zero)
            pltpu.async_copy(embs_hbm.at[idx_v], rows_v, sem).wait()
            pltpu.sync_copy(rows_v, seq_hbm.at[pl.ds(wid * epw, epw)])

    return k(bt_t, embs_flat, starts_t, aux)


def _tc_broadcast(seq_pad, tot):
    batch = 1024
    bt_tile = 32
    nbt = batch // bt_tile

    def body(tot_ref, seq_ref, out_ref):
        b = pl.program_id(0)
        thr = L - tot_ref[b]
        row = lax.broadcasted_iota(jnp.int32, (1, 1, L, H), 2)
        s = seq_ref[:, :L, :][:, None, :, :]
        s = jnp.where(row >= thr, s, 0.0)
        out_ref[...] = jnp.broadcast_to(s, (1, bt_tile, L, H))

    return pl.pallas_call(
        body,
        grid=(BTYPES, nbt),
        in_specs=[
            pl.BlockSpec(memory_space=pltpu.SMEM),
            pl.BlockSpec((1, SLOTS, H), lambda b, j: (b, 0, 0)),
        ],
        out_specs=pl.BlockSpec((1, bt_tile, L, H), lambda b, j: (b, j, 0, 0)),
        out_shape=jax.ShapeDtypeStruct((BTYPES, batch, L, H), jnp.float32),
    )(tot, seq_pad)


def kernel(input_embs, input_bt):
    bb, ll, hh = input_embs.shape
    n_tok = bb * ll
    nsc = NSUB * NLANE
    sub_len = n_tok // nsc
    embs_flat = input_embs.reshape(-1, hh)
    bt_t = (input_bt.reshape(NSUB, NLANE, sub_len)
            .transpose(0, 2, 1).reshape(NSUB, sub_len * NLANE))

    counts = _sc_counts(bt_t)
    c256 = counts.transpose(1, 0, 2).reshape(BTYPES, nsc)
    starts256, aux = _tc_prefix(c256)
    starts_t = (starts256.reshape(BTYPES, NSUB, NLANE)
                .transpose(1, 0, 2))
    seq_pad = _sc_select(bt_t, embs_flat, starts_t, aux)
    tot = aux[2 * BTYPES:, 0]
    return _tc_broadcast(seq_pad.reshape(BTYPES, SLOTS, hh), tot)


# speedup vs baseline: 2.7245x; 1.0049x over previous
import functools

import jax
import jax.numpy as jnp
from jax import lax
from jax.experimental import pallas as pl
from jax.experimental.pallas import tpu as pltpu
from jax.experimental.pallas import tpu_sc as plsc

BTYPES = 4
L = 200
H = 64
NLANE = 16
NSUB = 16
SLOTS = 256
NBUF = BTYPES * SLOTS
CHK = 128


def _mesh():
    return plsc.VectorSubcoreMesh(
        core_axis_name="c", subcore_axis_name="s",
        num_cores=2, num_subcores=NSUB)


def _sc_counts(bt_t):
    chunk = bt_t.shape[1]
    sub_len = chunk // NLANE

    @functools.partial(
        pl.kernel,
        out_type=jax.ShapeDtypeStruct((NSUB, BTYPES, NLANE), jnp.int32),
        mesh=_mesh(),
        scratch_types=[
            pltpu.VMEM((chunk,), jnp.int32),
            pltpu.VMEM((BTYPES, NLANE), jnp.int32),
        ],
    )
    def k(bt_hbm, cnt_hbm, bt_v, cnt4_v):
        cid = lax.axis_index("c")
        wid = lax.axis_index("s")

        @pl.when(cid == 0)
        def _():
            zero = jnp.zeros((NLANE,), jnp.int32)
            one = jnp.full((NLANE,), 1, jnp.int32)
            pltpu.sync_copy(bt_hbm.at[wid], bt_v)

            def p1(i, acc):
                v = bt_v[pl.ds(i * NLANE, NLANE)]
                return tuple(acc[b] + jnp.where(v == (b + 1), one, zero)
                             for b in range(BTYPES))
            acc = lax.fori_loop(0, sub_len, p1, (zero,) * BTYPES)
            for b in range(BTYPES):
                cnt4_v[b] = acc[b]
            pltpu.sync_copy(cnt4_v, cnt_hbm.at[wid])

    return k(bt_t)


def _tc_prefix(c256):
    nsc = NSUB * NLANE

    def body(c_ref, st_ref, aux_ref):
        cf = c_ref[...].astype(jnp.float32)
        ri = lax.broadcasted_iota(jnp.int32, (nsc, nsc), 0)
        ci = lax.broadcasted_iota(jnp.int32, (nsc, nsc), 1)
        tmat = jnp.where(ri < ci, 1.0, 0.0)
        ex = jnp.dot(cf, tmat, preferred_element_type=jnp.float32)
        st_ref[...] = ex.astype(jnp.int32)
        tot = jnp.sum(c_ref[...], axis=1, keepdims=True)
        thr = jnp.maximum(tot - L, 0)
        off = (L - 1) - tot
        aux_ref[...] = jnp.concatenate(
            [jnp.broadcast_to(thr, (BTYPES, NLANE)),
             jnp.broadcast_to(off, (BTYPES, NLANE)),
             jnp.broadcast_to(tot, (BTYPES, NLANE))], axis=0)

    return pl.pallas_call(
        body,
        out_shape=(
            jax.ShapeDtypeStruct((BTYPES, nsc), jnp.int32),
            jax.ShapeDtypeStruct((3 * BTYPES, NLANE), jnp.int32),
        ),
    )(c256)


def _sc_select(bt_t, embs_flat, starts_t, aux):
    chunk = bt_t.shape[1]
    sub_len = chunk // NLANE
    epw = NBUF // NSUB
    nchunks = chunk // CHK
    vper = CHK // NLANE

    @functools.partial(
        pl.kernel,
        out_type=jax.ShapeDtypeStruct((NBUF, H), jnp.float32),
        mesh=_mesh(),
        compiler_params=pltpu.CompilerParams(use_tc_tiling_on_sc=False),
        scratch_types=[
            pltpu.VMEM((chunk,), jnp.int32),
            pltpu.VMEM((BTYPES, NLANE), jnp.int32),
            pltpu.VMEM((2 * BTYPES, NLANE), jnp.int32),
            pltpu.VMEM((BTYPES, chunk), jnp.int32),
            pltpu.VMEM((BTYPES, chunk), jnp.int32),
            pltpu.VMEM((NBUF,), jnp.int32),
            pltpu.VMEM((epw,), jnp.int32),
            pltpu.VMEM((epw,), jnp.int32),
            pltpu.VMEM((epw, H), jnp.float32),
            pltpu.VMEM_SHARED((NBUF,), jnp.int32),
            pltpu.SemaphoreType.DMA,
        ],
    )
    def k(bt_hbm, embs_hbm, st_hbm, aux_hbm, seq_hbm,
          bt_v, st_v, to_v, slot_v, val_v, zb_v, mg_v, idx_v, rows_v,
          msh, sem):
        cid = lax.axis_index("c")
        wid = lax.axis_index("s")

        @pl.when(cid == 0)
        def _():
            iota = lax.iota(jnp.int32, NLANE)
            zero = jnp.zeros((NLANE,), jnp.int32)
            one = jnp.full((NLANE,), 1, jnp.int32)

            for i in range(NBUF // NLANE):
                zb_v[pl.ds(i * NLANE, NLANE)] = zero

            @pl.when(wid == 0)
            def _():
                pltpu.sync_copy(zb_v, msh)

            pltpu.sync_copy(bt_hbm.at[wid], bt_v)
            pltpu.sync_copy(st_hbm.at[wid], st_v)
            pltpu.sync_copy(aux_hbm.at[pl.ds(0, 2 * BTYPES)], to_v)

            starts = [st_v[b] for b in range(BTYPES)]
            thr = [to_v[b] for b in range(BTYPES)]
            off = [to_v[BTYPES + b] for b in range(BTYPES)]
            pos_base = (wid * NLANE + iota) * sub_len + 1

            def p2(i, carry):
                v = bt_v[pl.ds(i * NLANE, NLANE)]
                pos1 = pos_base + i
                new = []
                for b in range(BTYPES):
                    mi = jnp.where(v == (b + 1), one, zero)
                    r = carry[b] + mi
                    ki = jnp.where(mi * r - thr[b] > zero, one, zero)
                    slot = jnp.maximum(r + off[b], zero) + b * SLOTS
                    val_v[b, pl.ds(i * NLANE, NLANE)] = ki * pos1
                    slot_v[b, pl.ds(i * NLANE, NLANE)] = slot
                    new.append(r)
                return tuple(new)

            lax.fori_loop(0, sub_len, p2, tuple(starts))

            plsc.subcore_barrier()

            def merge(j, carry):
                for b in range(BTYPES):
                    pltpu.async_copy(
                        val_v.at[b, pl.ds(j * CHK, CHK)],
                        msh.at[slot_v.at[b, pl.ds(j * CHK, CHK)]],
                        sem, add=True)
                return carry

            lax.fori_loop(0, nchunks, merge, 0)
            pltpu.make_async_copy(bt_hbm.at[pl.ds(0, BTYPES)], val_v,
                                  sem).wait()
            plsc.subcore_barrier()

            pltpu.sync_copy(msh.at[pl.ds(wid * epw, epw)], mg_v)
            for j in range(epw // NLANE):
                idx_v[pl.ds(j * NLANE, NLANE)] = jnp.maximum(
                    mg_v[pl.ds(j * NLANE, NLANE)] - one, zero)
            pltpu.async_copy(embs_hbm.at[idx_v], rows_v, sem).wait()
            pltpu.sync_copy(rows_v, seq_hbm.at[pl.ds(wid * epw, epw)])

    return k(bt_t, embs_flat, starts_t, aux)


def _tc_broadcast(seq_pad, tot):
    batch = 1024
    bt_tile = 32
    nbt = batch // bt_tile

    def body(tot_ref, seq_ref, out_ref):
        b = pl.program_id(0)
        thr = L - tot_ref[b]
        row = lax.broadcasted_iota(jnp.int32, (1, 1, L, H), 2)
        s = seq_ref[:, :L, :][:, None, :, :]
        s = jnp.where(row >= thr, s, 0.0)
        out_ref[...] = jnp.broadcast_to(s, (1, bt_tile, L, H))

    return pl.pallas_call(
        body,
        grid=(BTYPES, nbt),
        in_specs=[
            pl.BlockSpec(memory_space=pltpu.SMEM),
            pl.BlockSpec((1, SLOTS, H), lambda b, j: (b, 0, 0)),
        ],
        out_specs=pl.BlockSpec((1, bt_tile, L, H), lambda b, j: (b, j, 0, 0)),
        out_shape=jax.ShapeDtypeStruct((BTYPES, batch, L, H), jnp.float32),
    )(tot, seq_pad)


def kernel(input_embs, input_bt):
    bb, ll, hh = input_embs.shape
    n_tok = bb * ll
    nsc = NSUB * NLANE
    sub_len = n_tok // nsc
    embs_flat = input_embs.reshape(-1, hh)
    bt_t = (input_bt.reshape(NSUB, NLANE, sub_len)
            .transpose(0, 2, 1).reshape(NSUB, sub_len * NLANE))

    counts = _sc_counts(bt_t)
    c256 = counts.transpose(1, 0, 2).reshape(BTYPES, nsc)
    starts256, aux = _tc_prefix(c256)
    starts_t = (starts256.reshape(BTYPES, NSUB, NLANE)
                .transpose(1, 0, 2))
    seq_pad = _sc_select(bt_t, embs_flat, starts_t, aux)
    tot = aux[2 * BTYPES:, 0]
    return _tc_broadcast(seq_pad.reshape(BTYPES, SLOTS, hh), tot)


# speedup vs baseline: 4.0295x; 1.4790x over previous
import functools

import jax
import jax.numpy as jnp
from jax import lax
from jax.experimental import pallas as pl
from jax.experimental.pallas import tpu as pltpu
from jax.experimental.pallas import tpu_sc as plsc

BTYPES = 4
L = 200
H = 64
NLANE = 16
NSUB = 16
SLOTS = 256
NBUF = BTYPES * SLOTS
CHK = 128


def _mesh():
    return plsc.VectorSubcoreMesh(
        core_axis_name="c", subcore_axis_name="s",
        num_cores=2, num_subcores=NSUB)


def _sc_counts(bt_t):
    chunk = bt_t.shape[1]
    sub_len = chunk // NLANE

    @functools.partial(
        pl.kernel,
        out_type=jax.ShapeDtypeStruct((NSUB, BTYPES, NLANE), jnp.int32),
        mesh=_mesh(),
        scratch_types=[
            pltpu.VMEM((chunk,), jnp.int32),
            pltpu.VMEM((BTYPES, NLANE), jnp.int32),
        ],
    )
    def k(bt_hbm, cnt_hbm, bt_v, cnt4_v):
        cid = lax.axis_index("c")
        wid = lax.axis_index("s")

        @pl.when(cid == 0)
        def _():
            zero = jnp.zeros((NLANE,), jnp.int32)
            one = jnp.full((NLANE,), 1, jnp.int32)
            pltpu.sync_copy(bt_hbm.at[wid], bt_v)

            def p1(i, acc):
                v = bt_v[pl.ds(i * NLANE, NLANE)]
                return tuple(acc[b] + jnp.where(v == (b + 1), one, zero)
                             for b in range(BTYPES))
            acc = lax.fori_loop(0, sub_len, p1, (zero,) * BTYPES)
            for b in range(BTYPES):
                cnt4_v[b] = acc[b]
            pltpu.sync_copy(cnt4_v, cnt_hbm.at[wid])

    return k(bt_t)


def _tc_prefix(c256):
    nsc = NSUB * NLANE

    def body(c_ref, st_ref, aux_ref):
        cf = c_ref[...].astype(jnp.float32)
        ri = lax.broadcasted_iota(jnp.int32, (nsc, nsc), 0)
        ci = lax.broadcasted_iota(jnp.int32, (nsc, nsc), 1)
        tmat = jnp.where(ri < ci, 1.0, 0.0)
        ex = jnp.dot(cf, tmat, preferred_element_type=jnp.float32)
        st_ref[...] = ex.astype(jnp.int32)
        tot = jnp.sum(c_ref[...], axis=1, keepdims=True)
        thr = jnp.maximum(tot - L, 0)
        off = (L - 1) - tot
        aux_ref[...] = jnp.concatenate(
            [jnp.broadcast_to(thr, (BTYPES, NLANE)),
             jnp.broadcast_to(off, (BTYPES, NLANE)),
             jnp.broadcast_to(tot, (BTYPES, NLANE))], axis=0)

    return pl.pallas_call(
        body,
        out_shape=(
            jax.ShapeDtypeStruct((BTYPES, nsc), jnp.int32),
            jax.ShapeDtypeStruct((3 * BTYPES, NLANE), jnp.int32),
        ),
    )(c256)


def _sc_select(bt_t, embs_flat, starts_t, aux):
    chunk = bt_t.shape[1]
    sub_len = chunk // NLANE
    epw = NBUF // NSUB
    nchunks = chunk // CHK
    vper = CHK // NLANE

    @functools.partial(
        pl.kernel,
        out_type=jax.ShapeDtypeStruct((NBUF, H), jnp.float32),
        mesh=_mesh(),
        compiler_params=pltpu.CompilerParams(use_tc_tiling_on_sc=False),
        scratch_types=[
            pltpu.VMEM((chunk,), jnp.int32),
            pltpu.VMEM((BTYPES, NLANE), jnp.int32),
            pltpu.VMEM((2 * BTYPES, NLANE), jnp.int32),
            pltpu.VMEM((chunk,), jnp.int32),
            pltpu.VMEM((chunk,), jnp.int32),
            pltpu.VMEM((NBUF,), jnp.int32),
            pltpu.VMEM((epw,), jnp.int32),
            pltpu.VMEM((epw,), jnp.int32),
            pltpu.VMEM((epw, H), jnp.float32),
            pltpu.VMEM_SHARED((NBUF,), jnp.int32),
            pltpu.SemaphoreType.DMA,
        ],
    )
    def k(bt_hbm, embs_hbm, st_hbm, aux_hbm, seq_hbm,
          bt_v, st_v, to_v, slot_v, val_v, zb_v, mg_v, idx_v, rows_v,
          msh, sem):
        cid = lax.axis_index("c")
        wid = lax.axis_index("s")

        @pl.when(cid == 0)
        def _():
            iota = lax.iota(jnp.int32, NLANE)
            zero = jnp.zeros((NLANE,), jnp.int32)
            one = jnp.full((NLANE,), 1, jnp.int32)

            for i in range(NBUF // NLANE):
                zb_v[pl.ds(i * NLANE, NLANE)] = zero

            @pl.when(wid == 0)
            def _():
                pltpu.sync_copy(zb_v, msh)

            pltpu.sync_copy(bt_hbm.at[wid], bt_v)
            pltpu.sync_copy(st_hbm.at[wid], st_v)
            pltpu.sync_copy(aux_hbm.at[pl.ds(0, 2 * BTYPES)], to_v)

            starts = [st_v[b] for b in range(BTYPES)]
            thr = [to_v[b] for b in range(BTYPES)]
            off = [to_v[BTYPES + b] for b in range(BTYPES)]
            pos_base = (wid * NLANE + iota) * sub_len + 1

            def p2(i, carry):
                v = bt_v[pl.ds(i * NLANE, NLANE)]
                pos1 = pos_base + i
                new = []
                slotc = zero
                valc = zero
                for b in range(BTYPES):
                    mi = jnp.where(v == (b + 1), one, zero)
                    r = carry[b] + mi
                    ki = jnp.where(mi * r - thr[b] > zero, one, zero)
                    slot = jnp.maximum(r + off[b], zero) + b * SLOTS
                    slotc = slotc + mi * slot
                    valc = valc + ki
                    new.append(r)
                val_v[pl.ds(i * NLANE, NLANE)] = valc * pos1
                slot_v[pl.ds(i * NLANE, NLANE)] = slotc
                return tuple(new)

            lax.fori_loop(0, sub_len, p2, tuple(starts))

            plsc.subcore_barrier()

            def merge(j, carry):
                pltpu.async_copy(
                    val_v.at[pl.ds(j * CHK, CHK)],
                    msh.at[slot_v.at[pl.ds(j * CHK, CHK)]],
                    sem, add=True)
                return carry

            lax.fori_loop(0, nchunks, merge, 0)
            pltpu.make_async_copy(bt_hbm.at[wid], val_v, sem).wait()
            plsc.subcore_barrier()

            pltpu.sync_copy(msh.at[pl.ds(wid * epw, epw)], mg_v)
            for j in range(epw // NLANE):
                idx_v[pl.ds(j * NLANE, NLANE)] = jnp.maximum(
                    mg_v[pl.ds(j * NLANE, NLANE)] - one, zero)
            pltpu.async_copy(embs_hbm.at[idx_v], rows_v, sem).wait()
            pltpu.sync_copy(rows_v, seq_hbm.at[pl.ds(wid * epw, epw)])

    return k(bt_t, embs_flat, starts_t, aux)


def _tc_broadcast(seq_pad, tot):
    batch = 1024
    bt_tile = 32
    nbt = batch // bt_tile

    def body(tot_ref, seq_ref, out_ref):
        b = pl.program_id(0)
        thr = L - tot_ref[b]
        row = lax.broadcasted_iota(jnp.int32, (1, 1, L, H), 2)
        s = seq_ref[:, :L, :][:, None, :, :]
        s = jnp.where(row >= thr, s, 0.0)
        out_ref[...] = jnp.broadcast_to(s, (1, bt_tile, L, H))

    return pl.pallas_call(
        body,
        grid=(BTYPES, nbt),
        in_specs=[
            pl.BlockSpec(memory_space=pltpu.SMEM),
            pl.BlockSpec((1, SLOTS, H), lambda b, j: (b, 0, 0)),
        ],
        out_specs=pl.BlockSpec((1, bt_tile, L, H), lambda b, j: (b, j, 0, 0)),
        out_shape=jax.ShapeDtypeStruct((BTYPES, batch, L, H), jnp.float32),
    )(tot, seq_pad)


def kernel(input_embs, input_bt):
    bb, ll, hh = input_embs.shape
    n_tok = bb * ll
    nsc = NSUB * NLANE
    sub_len = n_tok // nsc
    embs_flat = input_embs.reshape(-1, hh)
    bt_t = (input_bt.reshape(NSUB, NLANE, sub_len)
            .transpose(0, 2, 1).reshape(NSUB, sub_len * NLANE))

    counts = _sc_counts(bt_t)
    c256 = counts.transpose(1, 0, 2).reshape(BTYPES, nsc)
    starts256, aux = _tc_prefix(c256)
    starts_t = (starts256.reshape(BTYPES, NSUB, NLANE)
                .transpose(1, 0, 2))
    seq_pad = _sc_select(bt_t, embs_flat, starts_t, aux)
    tot = aux[2 * BTYPES:, 0]
    return _tc_broadcast(seq_pad.reshape(BTYPES, SLOTS, hh), tot)


# speedup vs baseline: 4.0524x; 1.0057x over previous
import functools

import jax
import jax.numpy as jnp
from jax import lax
from jax.experimental import pallas as pl
from jax.experimental.pallas import tpu as pltpu
from jax.experimental.pallas import tpu_sc as plsc

BTYPES = 4
L = 200
H = 64
NLANE = 16
NSUB = 16
SLOTS = 256
NBUF = BTYPES * SLOTS
CHK = 128


def _mesh():
    return plsc.VectorSubcoreMesh(
        core_axis_name="c", subcore_axis_name="s",
        num_cores=2, num_subcores=NSUB)


def _sc_counts(bt_t):
    chunk = bt_t.shape[1]
    sub_len = chunk // NLANE

    @functools.partial(
        pl.kernel,
        out_type=jax.ShapeDtypeStruct((NSUB, BTYPES, NLANE), jnp.int32),
        mesh=_mesh(),
        scratch_types=[
            pltpu.VMEM((chunk,), jnp.int32),
            pltpu.VMEM((BTYPES, NLANE), jnp.int32),
        ],
    )
    def k(bt_hbm, cnt_hbm, bt_v, cnt4_v):
        cid = lax.axis_index("c")
        wid = lax.axis_index("s")

        @pl.when(cid == 0)
        def _():
            zero = jnp.zeros((NLANE,), jnp.int32)
            one = jnp.full((NLANE,), 1, jnp.int32)
            pltpu.sync_copy(bt_hbm.at[wid], bt_v)

            def p1(i, acc):
                v = bt_v[pl.ds(i * NLANE, NLANE)]
                return tuple(acc[b] + jnp.where(v == (b + 1), one, zero)
                             for b in range(BTYPES))
            acc = lax.fori_loop(0, sub_len, p1, (zero,) * BTYPES)
            for b in range(BTYPES):
                cnt4_v[b] = acc[b]
            pltpu.sync_copy(cnt4_v, cnt_hbm.at[wid])

    return k(bt_t)


def _tc_prefix(c256):
    nsc = NSUB * NLANE

    def body(c_ref, st_ref, aux_ref):
        cf = c_ref[...].astype(jnp.float32)
        ri = lax.broadcasted_iota(jnp.int32, (nsc, nsc), 0)
        ci = lax.broadcasted_iota(jnp.int32, (nsc, nsc), 1)
        tmat = jnp.where(ri < ci, 1.0, 0.0)
        ex = jnp.dot(cf, tmat, preferred_element_type=jnp.float32)
        st_ref[...] = ex.astype(jnp.int32)
        tot = jnp.sum(c_ref[...], axis=1, keepdims=True)
        thr = jnp.maximum(tot - L, 0)
        off = (L - 1) - tot
        aux_ref[...] = jnp.concatenate(
            [jnp.broadcast_to(thr, (BTYPES, NLANE)),
             jnp.broadcast_to(off, (BTYPES, NLANE)),
             jnp.broadcast_to(tot, (BTYPES, NLANE))], axis=0)

    return pl.pallas_call(
        body,
        out_shape=(
            jax.ShapeDtypeStruct((BTYPES, nsc), jnp.int32),
            jax.ShapeDtypeStruct((3 * BTYPES, NLANE), jnp.int32),
        ),
    )(c256)


def _sc_select(bt_t, embs_flat, starts_t, aux):
    chunk = bt_t.shape[1]
    sub_len = chunk // NLANE
    epw = NBUF // NSUB
    nchunks = chunk // CHK
    vper = CHK // NLANE

    @functools.partial(
        pl.kernel,
        out_type=jax.ShapeDtypeStruct((NBUF, H), jnp.float32),
        mesh=_mesh(),
        compiler_params=pltpu.CompilerParams(use_tc_tiling_on_sc=False),
        scratch_types=[
            pltpu.VMEM((chunk,), jnp.int32),
            pltpu.VMEM((BTYPES, NLANE), jnp.int32),
            pltpu.VMEM((2 * BTYPES, NLANE), jnp.int32),
            pltpu.VMEM((chunk,), jnp.int32),
            pltpu.VMEM((chunk,), jnp.int32),
            pltpu.VMEM((NBUF,), jnp.int32),
            pltpu.VMEM((epw,), jnp.int32),
            pltpu.VMEM((epw,), jnp.int32),
            pltpu.VMEM((epw, H), jnp.float32),
            pltpu.VMEM_SHARED((NBUF,), jnp.int32),
            pltpu.SemaphoreType.DMA,
        ],
    )
    def k(bt_hbm, embs_hbm, st_hbm, aux_hbm, seq_hbm,
          bt_v, st_v, to_v, slot_v, val_v, zb_v, mg_v, idx_v, rows_v,
          msh, sem):
        cid = lax.axis_index("c")
        wid = lax.axis_index("s")

        @pl.when(cid == 0)
        def _():
            iota = lax.iota(jnp.int32, NLANE)
            zero = jnp.zeros((NLANE,), jnp.int32)
            one = jnp.full((NLANE,), 1, jnp.int32)

            for i in range(NBUF // NLANE):
                zb_v[pl.ds(i * NLANE, NLANE)] = zero

            @pl.when(wid == 0)
            def _():
                pltpu.sync_copy(zb_v, msh)

            pltpu.sync_copy(bt_hbm.at[wid], bt_v)
            pltpu.sync_copy(st_hbm.at[wid], st_v)
            pltpu.sync_copy(aux_hbm.at[pl.ds(0, 2 * BTYPES)], to_v)

            starts = [st_v[b] for b in range(BTYPES)]
            thr = [to_v[b] for b in range(BTYPES)]
            off = [to_v[BTYPES + b] for b in range(BTYPES)]
            pos_base = (wid * NLANE + iota) * sub_len + 1

            def p2(i, carry):
                v = bt_v[pl.ds(i * NLANE, NLANE)]
                pos1 = pos_base + i
                new = []
                slotc = zero
                valc = zero
                for b in range(BTYPES):
                    mi = jnp.where(v == (b + 1), one, zero)
                    r = carry[b] + mi
                    ki = jnp.where(mi * r - thr[b] > zero, one, zero)
                    slot = jnp.maximum(r + off[b], zero) + b * SLOTS
                    slotc = slotc + mi * slot
                    valc = valc + ki
                    new.append(r)
                val_v[pl.ds(i * NLANE, NLANE)] = valc * pos1
                slot_v[pl.ds(i * NLANE, NLANE)] = slotc
                return tuple(new)

            lax.fori_loop(0, sub_len, p2, tuple(starts))

            plsc.subcore_barrier()

            def merge(j, carry):
                pltpu.async_copy(
                    val_v.at[pl.ds(j * CHK, CHK)],
                    msh.at[slot_v.at[pl.ds(j * CHK, CHK)]],
                    sem, add=True)
                return carry

            lax.fori_loop(0, nchunks, merge, 0)
            pltpu.make_async_copy(bt_hbm.at[wid], val_v, sem).wait()
            plsc.subcore_barrier()

            pltpu.sync_copy(msh.at[pl.ds(wid * epw, epw)], mg_v)
            for j in range(epw // NLANE):
                idx_v[pl.ds(j * NLANE, NLANE)] = jnp.maximum(
                    mg_v[pl.ds(j * NLANE, NLANE)] - one, zero)
            pltpu.async_copy(embs_hbm.at[idx_v], rows_v, sem).wait()
            pltpu.sync_copy(rows_v, seq_hbm.at[pl.ds(wid * epw, epw)])

    return k(bt_t, embs_flat, starts_t, aux)


def _tc_broadcast(seq_pad, tot):
    batch = 1024
    bt_tile = 64
    nbt = batch // bt_tile

    def body(tot_ref, seq_ref, out_ref):
        b = pl.program_id(0)
        thr = L - tot_ref[b]
        row = lax.broadcasted_iota(jnp.int32, (1, L, H), 1)
        s = jnp.where(row >= thr, seq_ref[:, :L, :], 0.0)
        out_ref[...] = jnp.broadcast_to(s[:, None, :, :],
                                        (1, bt_tile, L, H))

    return pl.pallas_call(
        body,
        grid=(BTYPES, nbt),
        in_specs=[
            pl.BlockSpec(memory_space=pltpu.SMEM),
            pl.BlockSpec((1, SLOTS, H), lambda b, j: (b, 0, 0)),
        ],
        out_specs=pl.BlockSpec((1, bt_tile, L, H), lambda b, j: (b, j, 0, 0)),
        out_shape=jax.ShapeDtypeStruct((BTYPES, batch, L, H), jnp.float32),
    )(tot, seq_pad)


def kernel(input_embs, input_bt):
    bb, ll, hh = input_embs.shape
    n_tok = bb * ll
    nsc = NSUB * NLANE
    sub_len = n_tok // nsc
    embs_flat = input_embs.reshape(-1, hh)
    bt_t = (input_bt.reshape(NSUB, NLANE, sub_len)
            .transpose(0, 2, 1).reshape(NSUB, sub_len * NLANE))

    counts = _sc_counts(bt_t)
    c256 = counts.transpose(1, 0, 2).reshape(BTYPES, nsc)
    starts256, aux = _tc_prefix(c256)
    starts_t = (starts256.reshape(BTYPES, NSUB, NLANE)
                .transpose(1, 0, 2))
    seq_pad = _sc_select(bt_t, embs_flat, starts_t, aux)
    tot = aux[2 * BTYPES:, 0]
    return _tc_broadcast(seq_pad.reshape(BTYPES, SLOTS, hh), tot)


# speedup vs baseline: 4.0566x; 1.0010x over previous
import functools

import jax
import jax.numpy as jnp
from jax import lax
from jax.experimental import pallas as pl
from jax.experimental.pallas import tpu as pltpu
from jax.experimental.pallas import tpu_sc as plsc

BTYPES = 4
L = 200
H = 64
NLANE = 16
NSUB = 16
SLOTS = 256
NBUF = BTYPES * SLOTS
CHK = 128


def _mesh():
    return plsc.VectorSubcoreMesh(
        core_axis_name="c", subcore_axis_name="s",
        num_cores=2, num_subcores=NSUB)


def _sc_counts(bt_t):
    chunk = bt_t.shape[1]
    sub_len = chunk // NLANE

    @functools.partial(
        pl.kernel,
        out_type=jax.ShapeDtypeStruct((NSUB, BTYPES, NLANE), jnp.int32),
        mesh=_mesh(),
        scratch_types=[
            pltpu.VMEM((chunk,), jnp.int32),
            pltpu.VMEM((BTYPES, NLANE), jnp.int32),
        ],
    )
    def k(bt_hbm, cnt_hbm, bt_v, cnt4_v):
        cid = lax.axis_index("c")
        wid = lax.axis_index("s")

        @pl.when(cid == 0)
        def _():
            zero = jnp.zeros((NLANE,), jnp.int32)
            one = jnp.full((NLANE,), 1, jnp.int32)
            pltpu.sync_copy(bt_hbm.at[wid], bt_v)

            def p1(i, acc):
                v = bt_v[pl.ds(i * NLANE, NLANE)]
                return tuple(acc[b] + jnp.where(v == (b + 1), one, zero)
                             for b in range(BTYPES))
            acc = lax.fori_loop(0, sub_len, p1, (zero,) * BTYPES)
            for b in range(BTYPES):
                cnt4_v[b] = acc[b]
            pltpu.sync_copy(cnt4_v, cnt_hbm.at[wid])

    return k(bt_t)


def _tc_prefix(c256):
    nsc = NSUB * NLANE

    def body(c_ref, st_ref, aux_ref):
        cf = c_ref[...].astype(jnp.float32)
        ri = lax.broadcasted_iota(jnp.int32, (nsc, nsc), 0)
        ci = lax.broadcasted_iota(jnp.int32, (nsc, nsc), 1)
        tmat = jnp.where(ri < ci, 1.0, 0.0)
        ex = jnp.dot(cf, tmat, preferred_element_type=jnp.float32)
        st_ref[...] = ex.astype(jnp.int32)
        tot = jnp.sum(c_ref[...], axis=1, keepdims=True)
        thr = jnp.maximum(tot - L, 0)
        off = (L - 1) - tot
        aux_ref[...] = jnp.concatenate(
            [jnp.broadcast_to(thr, (BTYPES, NLANE)),
             jnp.broadcast_to(off, (BTYPES, NLANE)),
             jnp.broadcast_to(tot, (BTYPES, NLANE))], axis=0)

    return pl.pallas_call(
        body,
        out_shape=(
            jax.ShapeDtypeStruct((BTYPES, nsc), jnp.int32),
            jax.ShapeDtypeStruct((3 * BTYPES, NLANE), jnp.int32),
        ),
    )(c256)


def _sc_select(bt_t, embs_flat, starts_t, aux):
    chunk = bt_t.shape[1]
    sub_len = chunk // NLANE
    epw = NBUF // NSUB
    nchunks = chunk // CHK
    vper = CHK // NLANE

    @functools.partial(
        pl.kernel,
        out_type=jax.ShapeDtypeStruct((NBUF, H), jnp.float32),
        mesh=_mesh(),
        compiler_params=pltpu.CompilerParams(use_tc_tiling_on_sc=False),
        scratch_types=[
            pltpu.VMEM((chunk,), jnp.int32),
            pltpu.VMEM((BTYPES, NLANE), jnp.int32),
            pltpu.VMEM((2 * BTYPES, NLANE), jnp.int32),
            pltpu.VMEM((chunk,), jnp.int32),
            pltpu.VMEM((chunk,), jnp.int32),
            pltpu.VMEM((NBUF,), jnp.int32),
            pltpu.VMEM((epw,), jnp.int32),
            pltpu.VMEM((epw,), jnp.int32),
            pltpu.VMEM((epw, H), jnp.float32),
            pltpu.VMEM_SHARED((NBUF,), jnp.int32),
            pltpu.SemaphoreType.DMA,
        ],
    )
    def k(bt_hbm, embs_hbm, st_hbm, aux_hbm, seq_hbm,
          bt_v, st_v, to_v, slot_v, val_v, zb_v, mg_v, idx_v, rows_v,
          msh, sem):
        cid = lax.axis_index("c")
        wid = lax.axis_index("s")

        @pl.when(cid == 0)
        def _():
            iota = lax.iota(jnp.int32, NLANE)
            zero = jnp.zeros((NLANE,), jnp.int32)
            one = jnp.full((NLANE,), 1, jnp.int32)

            for i in range(NBUF // NLANE):
                zb_v[pl.ds(i * NLANE, NLANE)] = zero

            @pl.when(wid == 0)
            def _():
                pltpu.sync_copy(zb_v, msh)

            pltpu.sync_copy(bt_hbm.at[wid], bt_v)
            pltpu.sync_copy(st_hbm.at[wid], st_v)
            pltpu.sync_copy(aux_hbm.at[pl.ds(0, 2 * BTYPES)], to_v)

            starts = [st_v[b] for b in range(BTYPES)]
            off2 = [to_v[BTYPES + b] + (b * SLOTS + 1) for b in range(BTYPES)]
            floor = [jnp.full((NLANE,), b * SLOTS, jnp.int32)
                     for b in range(BTYPES)]
            pos_base = (wid * NLANE + iota) * sub_len + 1

            def step(i, carry):
                v = bt_v[pl.ds(i * NLANE, NLANE)]
                pos1 = pos_base + i
                new = []
                slotc = zero
                valc = zero
                for b in range(BTYPES):
                    mi = jnp.where(v == (b + 1), one, zero)
                    r = carry[b] + mi
                    sraw = r + off2[b]
                    ki = jnp.where(mi * sraw - floor[b] > zero, one, zero)
                    slot = jnp.maximum(sraw - 1, floor[b])
                    slotc = slotc + mi * slot
                    valc = valc + ki
                    new.append(r)
                val_v[pl.ds(i * NLANE, NLANE)] = valc * pos1
                slot_v[pl.ds(i * NLANE, NLANE)] = slotc
                return tuple(new)

            def p2(i2, carry):
                carry = step(i2 * 2, carry)
                return step(i2 * 2 + 1, carry)

            lax.fori_loop(0, sub_len // 2, p2, tuple(starts))

            plsc.subcore_barrier()

            def merge(j, carry):
                pltpu.async_copy(
                    val_v.at[pl.ds(j * CHK, CHK)],
                    msh.at[slot_v.at[pl.ds(j * CHK, CHK)]],
                    sem, add=True)
                return carry

            lax.fori_loop(0, nchunks, merge, 0)
            pltpu.make_async_copy(bt_hbm.at[wid], val_v, sem).wait()
            plsc.subcore_barrier()

            pltpu.sync_copy(msh.at[pl.ds(wid * epw, epw)], mg_v)
            for j in range(epw // NLANE):
                idx_v[pl.ds(j * NLANE, NLANE)] = jnp.maximum(
                    mg_v[pl.ds(j * NLANE, NLANE)] - one, zero)
            pltpu.async_copy(embs_hbm.at[idx_v], rows_v, sem).wait()
            pltpu.sync_copy(rows_v, seq_hbm.at[pl.ds(wid * epw, epw)])

    return k(bt_t, embs_flat, starts_t, aux)


def _tc_broadcast(seq_pad, tot):
    batch = 1024
    bt_tile = 64
    nbt = batch // bt_tile

    def body(tot_ref, seq_ref, out_ref):
        b = pl.program_id(0)
        thr = L - tot_ref[b]
        row = lax.broadcasted_iota(jnp.int32, (1, L, H), 1)
        s = jnp.where(row >= thr, seq_ref[:, :L, :], 0.0)
        out_ref[...] = jnp.broadcast_to(s[:, None, :, :],
                                        (1, bt_tile, L, H))

    return pl.pallas_call(
        body,
        grid=(BTYPES, nbt),
        in_specs=[
            pl.BlockSpec(memory_space=pltpu.SMEM),
            pl.BlockSpec((1, SLOTS, H), lambda b, j: (b, 0, 0)),
        ],
        out_specs=pl.BlockSpec((1, bt_tile, L, H), lambda b, j: (b, j, 0, 0)),
        out_shape=jax.ShapeDtypeStruct((BTYPES, batch, L, H), jnp.float32),
    )(tot, seq_pad)


def kernel(input_embs, input_bt):
    bb, ll, hh = input_embs.shape
    n_tok = bb * ll
    nsc = NSUB * NLANE
    sub_len = n_tok // nsc
    embs_flat = input_embs.reshape(-1, hh)
    bt_t = (input_bt.reshape(NSUB, NLANE, sub_len)
            .transpose(0, 2, 1).reshape(NSUB, sub_len * NLANE))

    counts = _sc_counts(bt_t)
    c256 = counts.transpose(1, 0, 2).reshape(BTYPES, nsc)
    starts256, aux = _tc_prefix(c256)
    starts_t = (starts256.reshape(BTYPES, NSUB, NLANE)
                .transpose(1, 0, 2))
    seq_pad = _sc_select(bt_t, embs_flat, starts_t, aux)
    tot = aux[2 * BTYPES:, 0]
    return _tc_broadcast(seq_pad.reshape(BTYPES, SLOTS, hh), tot)
